# baseline (speedup 1.0000x reference)
import jax
import jax.numpy as jnp
from jax.experimental import pallas as pl
from jax.experimental.pallas import tpu as pltpu

EPS = 1e-5
WINDOW = 5
CHUNK = 256
HALO = 8


def _shift(x, d):
    if d == 0:
        return x
    cols = x.shape[1]
    if d > 0:
        return jnp.concatenate(
            [x[d:], jnp.zeros((d, cols), x.dtype)], axis=0)
    return jnp.concatenate(
        [jnp.zeros((-d, cols), x.dtype), x[:d]], axis=0)


def _body(cmax_ref, hp_ref, hm_ref, hn_ref, m_ref, tokw_ref, tokb_ref,
          clsw_ref, clsb_ref, lntg_ref, lntb_ref, lncg_ref, lncb_ref,
          cls_ref, reps_ref):
    del cmax_ref
    c = pl.program_id(1)
    S = m_ref.shape[-1]
    TD = tokw_ref.shape[1]
    W = WINDOW

    hh = jnp.concatenate([hp_ref[0], hm_ref[0], hn_ref[0]], axis=0)
    R = hh.shape[0]

    @pl.when(c == 0)
    def _():
        cb = jnp.dot(hh[HALO:HALO + 1, :].astype(jnp.bfloat16), clsw_ref[...],
                     preferred_element_type=jnp.float32) + clsb_ref[...]
        cm = jnp.mean(cb, axis=-1, keepdims=True)
        cv = jnp.mean((cb - cm) ** 2, axis=-1, keepdims=True)
        cls_ref[0] = ((cb - cm) * jax.lax.rsqrt(cv + EPS) * lncg_ref[...]
                      + lncb_ref[...])

    t = jnp.dot(hh.astype(jnp.bfloat16), tokw_ref[...],
                preferred_element_type=jnp.float32) + tokb_ref[...]
    tm = jnp.mean(t, axis=-1, keepdims=True)
    tv = jnp.mean((t - tm) ** 2, axis=-1, keepdims=True)
    t = (t - tm) * jax.lax.rsqrt(tv + EPS) * lntg_ref[...] + lntb_ref[...]
    r = jnp.maximum(t, 0.0)

    mv = m_ref[0]
    lane = jax.lax.broadcasted_iota(jnp.int32, (1, S), 1)
    L = jnp.sum(jnp.where((lane >= 1) & (lane < S - 1), mv, 0))

    q = (jax.lax.broadcasted_iota(jnp.int32, (R, 1), 0)
         + (c * CHUNK - HALO))
    rm = jnp.where((q >= 0) & (q < L), _shift(r, 1), 0.0)

    t2 = rm + _shift(rm, 1)
    t4 = t2 + _shift(t2, 2)
    t8 = t4 + _shift(t4, 4)
    ws = _shift(t8, -W) + _shift(t2, W - 2)

    n2 = jnp.sum(ws * ws, axis=-1, keepdims=True)
    scale = jnp.where((q < L) & (n2 > 0), jax.lax.rsqrt(n2), 0.0)
    reps_ref[0] = (ws * scale)[HALO:HALO + CHUNK]


def kernel(hidden, attention_mask, tok_w, tok_b, cls_w, cls_b,
           ln_tok_g, ln_tok_b, ln_cls_g, ln_cls_b):
    B, S, H = hidden.shape
    TD = tok_w.shape[1]
    CD = cls_w.shape[1]
    NC = S // CHUNK
    HB = CHUNK // HALO

    lens = jnp.sum(attention_mask[:, 1:S - 1], axis=1).astype(jnp.int32)
    cmax = jnp.clip((lens - 1) // CHUNK, 0, NC - 1).astype(jnp.int32)

    mask3 = attention_mask.reshape(B, 1, S)
    full = lambda shape: pl.BlockSpec(shape, lambda b, c, cm: (0,) * len(shape))

    def im_prev(b, c, cm):
        cc = jnp.minimum(c, cm[b])
        return (b, jnp.maximum(cc * HB - 1, 0), 0)

    def im_main(b, c, cm):
        return (b, jnp.minimum(c, cm[b]), 0)

    def im_next(b, c, cm):
        cc = jnp.minimum(c, cm[b])
        return (b, jnp.minimum(cc * HB + HB, S // HALO - 1), 0)

    grid_spec = pltpu.PrefetchScalarGridSpec(
        num_scalar_prefetch=1,
        grid=(B, NC),
        in_specs=[
            pl.BlockSpec((1, HALO, H), im_prev),
            pl.BlockSpec((1, CHUNK, H), im_main),
            pl.BlockSpec((1, HALO, H), im_next),
            pl.BlockSpec((1, 1, S), lambda b, c, cm: (b, 0, 0)),
            full((H, TD)),
            full((1, TD)),
            full((H, CD)),
            full((1, CD)),
            full((1, TD)),
            full((1, TD)),
            full((1, CD)),
            full((1, CD)),
        ],
        out_specs=[
            pl.BlockSpec((1, 1, CD), lambda b, c, cm: (b, 0, 0)),
            pl.BlockSpec((1, CHUNK, TD), lambda b, c, cm: (b, c, 0)),
        ],
    )

    cls3, reps = pl.pallas_call(
        _body,
        grid_spec=grid_spec,
        out_shape=[
            jax.ShapeDtypeStruct((B, 1, CD), jnp.float32),
            jax.ShapeDtypeStruct((B, S - 2, TD), jnp.float32),
        ],
        compiler_params=pltpu.CompilerParams(
            dimension_semantics=("parallel", "arbitrary"),
        ),
        name="coil_core_fused",
    )(cmax, hidden, hidden, hidden, mask3,
      tok_w.astype(jnp.bfloat16), tok_b.reshape(1, TD),
      cls_w.astype(jnp.bfloat16), cls_b.reshape(1, CD),
      ln_tok_g.reshape(1, TD), ln_tok_b.reshape(1, TD),
      ln_cls_g.reshape(1, CD), ln_cls_b.reshape(1, CD))

    return (cls3.reshape(B, CD), reps)


# speedup vs baseline: 2.3208x; 2.3208x over previous
import jax
import jax.numpy as jnp
from jax.experimental import pallas as pl
from jax.experimental.pallas import tpu as pltpu

EPS = 1e-5
WINDOW = 5


def _shift(x, d):
    if d == 0:
        return x
    cols = x.shape[1]
    if d > 0:
        return jnp.concatenate(
            [x[d:], jnp.zeros((d, cols), x.dtype)], axis=0)
    return jnp.concatenate(
        [jnp.zeros((-d, cols), x.dtype), x[:d]], axis=0)


def _body(h_ref, m_ref, tokw_ref, tokb_ref, clsw_ref, clsb_ref,
          lntg_ref, lntb_ref, lncg_ref, lncb_ref, cls_ref, reps_ref):
    h = h_ref[0]
    S = h.shape[0]
    TD = tokw_ref.shape[1]
    W = WINDOW

    c = jnp.dot(h[0:1, :].astype(jnp.bfloat16),
                clsw_ref[...].astype(jnp.bfloat16),
                preferred_element_type=jnp.float32) + clsb_ref[...]
    cm = jnp.mean(c, axis=-1, keepdims=True)
    cv = jnp.mean((c - cm) ** 2, axis=-1, keepdims=True)
    cls_ref[0] = (c - cm) * jax.lax.rsqrt(cv + EPS) * lncg_ref[...] + lncb_ref[...]

    t = jnp.dot(h.astype(jnp.bfloat16), tokw_ref[...].astype(jnp.bfloat16),
                preferred_element_type=jnp.float32) + tokb_ref[...]
    tm = jnp.mean(t, axis=-1, keepdims=True)
    tv = jnp.mean((t - tm) ** 2, axis=-1, keepdims=True)
    t = (t - tm) * jax.lax.rsqrt(tv + EPS) * lntg_ref[...] + lntb_ref[...]
    r = jnp.maximum(t, 0.0)

    mv = m_ref[0]
    lane = jax.lax.broadcasted_iota(jnp.int32, (1, S), 1)
    L = jnp.sum(jnp.where((lane >= 1) & (lane < S - 1), mv, 0))

    q = jax.lax.broadcasted_iota(jnp.int32, (S, 1), 0)
    rm = jnp.where(q < L,
                   jnp.concatenate([r[1:], jnp.zeros((1, TD), r.dtype)], axis=0),
                   0.0)

    rp = jnp.concatenate([jnp.zeros((8, TD), rm.dtype), rm], axis=0)
    t2 = rp + _shift(rp, 1)
    t4 = t2 + _shift(t2, 2)
    t8 = t4 + _shift(t4, 4)
    ws = (_shift(t8, -W) + _shift(t2, W - 2))[8:]

    n2 = jnp.sum(ws * ws, axis=-1, keepdims=True)
    scale = jnp.where((q < L) & (n2 > 0), jax.lax.rsqrt(n2), 0.0)
    reps_ref[0] = (ws * scale)[:S - 2]


def kernel(hidden, attention_mask, tok_w, tok_b, cls_w, cls_b,
           ln_tok_g, ln_tok_b, ln_cls_g, ln_cls_b):
    B, S, H = hidden.shape
    TD = tok_w.shape[1]
    CD = cls_w.shape[1]

    mask3 = attention_mask.reshape(B, 1, S)
    full = lambda shape: pl.BlockSpec(shape, lambda b: (0,) * len(shape))

    cls3, reps = pl.pallas_call(
        _body,
        grid=(B,),
        in_specs=[
            pl.BlockSpec((1, S, H), lambda b: (b, 0, 0)),
            pl.BlockSpec((1, 1, S), lambda b: (b, 0, 0)),
            full((H, TD)),
            full((1, TD)),
            full((H, CD)),
            full((1, CD)),
            full((1, TD)),
            full((1, TD)),
            full((1, CD)),
            full((1, CD)),
        ],
        out_specs=[
            pl.BlockSpec((1, 1, CD), lambda b: (b, 0, 0)),
            pl.BlockSpec((1, S - 2, TD), lambda b: (b, 0, 0)),
        ],
        out_shape=[
            jax.ShapeDtypeStruct((B, 1, CD), jnp.float32),
            jax.ShapeDtypeStruct((B, S - 2, TD), jnp.float32),
        ],
        compiler_params=pltpu.CompilerParams(
            dimension_semantics=("parallel",),
        ),
        name="coil_core_fused",
    )(hidden, mask3, tok_w, tok_b.reshape(1, TD),
      cls_w, cls_b.reshape(1, CD),
      ln_tok_g.reshape(1, TD), ln_tok_b.reshape(1, TD),
      ln_cls_g.reshape(1, CD), ln_cls_b.reshape(1, CD))

    return (cls3.reshape(B, CD), reps)


# speedup vs baseline: 2.3371x; 1.0070x over previous
import jax
import jax.numpy as jnp
from jax.experimental import pallas as pl
from jax.experimental.pallas import tpu as pltpu

EPS = 1e-5
WINDOW = 5


def _shift(x, d):
    if d == 0:
        return x
    cols = x.shape[1]
    if d > 0:
        return jnp.concatenate(
            [x[d:], jnp.zeros((d, cols), x.dtype)], axis=0)
    return jnp.concatenate(
        [jnp.zeros((-d, cols), x.dtype), x[:d]], axis=0)


def _body(h0_ref, h1_ref, h2_ref, h3_ref, m_ref, tokw_ref, tokb_ref,
          clsw_ref, clsb_ref, lntg_ref, lntb_ref, lncg_ref, lncb_ref,
          cls_ref, reps_ref):
    h_parts = (h0_ref[0], h1_ref[0], h2_ref[0], h3_ref[0])
    S = sum(p.shape[0] for p in h_parts)
    TD = tokw_ref.shape[1]
    W = WINDOW

    c = jnp.dot(h_parts[0][0:1, :].astype(jnp.bfloat16),
                clsw_ref[...].astype(jnp.bfloat16),
                preferred_element_type=jnp.float32) + clsb_ref[...]
    cm = jnp.mean(c, axis=-1, keepdims=True)
    cv = jnp.mean((c - cm) ** 2, axis=-1, keepdims=True)
    cls_ref[0] = (c - cm) * jax.lax.rsqrt(cv + EPS) * lncg_ref[...] + lncb_ref[...]

    tw = tokw_ref[...].astype(jnp.bfloat16)

    def proj(hp):
        t = jnp.dot(hp.astype(jnp.bfloat16), tw,
                    preferred_element_type=jnp.float32) + tokb_ref[...]
        tm = jnp.mean(t, axis=-1, keepdims=True)
        tv = jnp.mean((t - tm) ** 2, axis=-1, keepdims=True)
        t = (t - tm) * jax.lax.rsqrt(tv + EPS) * lntg_ref[...] + lntb_ref[...]
        return jnp.maximum(t, 0.0)

    r = jnp.concatenate([proj(hp) for hp in h_parts], axis=0)

    mv = m_ref[0]
    lane = jax.lax.broadcasted_iota(jnp.int32, (1, S), 1)
    L = jnp.sum(jnp.where((lane >= 1) & (lane < S - 1), mv, 0))

    q = jax.lax.broadcasted_iota(jnp.int32, (S, 1), 0)
    rm = jnp.where(q < L,
                   jnp.concatenate([r[1:], jnp.zeros((1, TD), r.dtype)], axis=0),
                   0.0)

    rp = jnp.concatenate([jnp.zeros((8, TD), rm.dtype), rm], axis=0)
    t2 = rp + _shift(rp, 1)
    t4 = t2 + _shift(t2, 2)
    t8 = t4 + _shift(t4, 4)
    ws = (_shift(t8, -W) + _shift(t2, W - 2))[8:]

    n2 = jnp.sum(ws * ws, axis=-1, keepdims=True)
    scale = jnp.where((q < L) & (n2 > 0), jax.lax.rsqrt(n2), 0.0)
    reps_ref[0] = (ws * scale)[:S - 2]


def kernel(hidden, attention_mask, tok_w, tok_b, cls_w, cls_b,
           ln_tok_g, ln_tok_b, ln_cls_g, ln_cls_b):
    B, S, H = hidden.shape
    TD = tok_w.shape[1]
    CD = cls_w.shape[1]

    mask3 = attention_mask.reshape(B, 1, S)
    full = lambda shape: pl.BlockSpec(shape, lambda b: (0,) * len(shape))

    cls3, reps = pl.pallas_call(
        _body,
        grid=(B,),
        in_specs=[
            pl.BlockSpec((1, S // 4, H), lambda b: (b, 0, 0)),
            pl.BlockSpec((1, S // 4, H), lambda b: (b, 1, 0)),
            pl.BlockSpec((1, S // 4, H), lambda b: (b, 2, 0)),
            pl.BlockSpec((1, S // 4, H), lambda b: (b, 3, 0)),
            pl.BlockSpec((1, 1, S), lambda b: (b, 0, 0)),
            full((H, TD)),
            full((1, TD)),
            full((H, CD)),
            full((1, CD)),
            full((1, TD)),
            full((1, TD)),
            full((1, CD)),
            full((1, CD)),
        ],
        out_specs=[
            pl.BlockSpec((1, 1, CD), lambda b: (b, 0, 0)),
            pl.BlockSpec((1, S - 2, TD), lambda b: (b, 0, 0)),
        ],
        out_shape=[
            jax.ShapeDtypeStruct((B, 1, CD), jnp.float32),
            jax.ShapeDtypeStruct((B, S - 2, TD), jnp.float32),
        ],
        compiler_params=pltpu.CompilerParams(
            dimension_semantics=("parallel",),
        ),
        name="coil_core_fused",
    )(hidden, hidden, hidden, hidden, mask3, tok_w, tok_b.reshape(1, TD),
      cls_w, cls_b.reshape(1, CD),
      ln_tok_g.reshape(1, TD), ln_tok_b.reshape(1, TD),
      ln_cls_g.reshape(1, CD), ln_cls_b.reshape(1, CD))

    return (cls3.reshape(B, CD), reps)


# speedup vs baseline: 2.3587x; 1.0092x over previous
import jax
import jax.numpy as jnp
from jax.experimental import pallas as pl
from jax.experimental.pallas import tpu as pltpu

EPS = 1e-5
WINDOW = 5


def _shift(x, d):
    if d == 0:
        return x
    cols = x.shape[1]
    if d > 0:
        return jnp.concatenate(
            [x[d:], jnp.zeros((d, cols), x.dtype)], axis=0)
    return jnp.concatenate(
        [jnp.zeros((-d, cols), x.dtype), x[:d]], axis=0)


def _body(h0_ref, h1_ref, h2_ref, h3_ref, h4_ref, h5_ref, h6_ref, h7_ref,
          m_ref, tokw_ref, tokb_ref,
          clsw_ref, clsb_ref, lntg_ref, lntb_ref, lncg_ref, lncb_ref,
          cls_ref, reps_ref):
    h_parts = (h0_ref[0], h1_ref[0], h2_ref[0], h3_ref[0],
               h4_ref[0], h5_ref[0], h6_ref[0], h7_ref[0])
    S = sum(p.shape[0] for p in h_parts)
    TD = tokw_ref.shape[1]
    W = WINDOW

    c = jnp.dot(h_parts[0][0:1, :].astype(jnp.bfloat16),
                clsw_ref[...].astype(jnp.bfloat16),
                preferred_element_type=jnp.float32) + clsb_ref[...]
    cm = jnp.mean(c, axis=-1, keepdims=True)
    cv = jnp.mean((c - cm) ** 2, axis=-1, keepdims=True)
    cls_ref[0] = (c - cm) * jax.lax.rsqrt(cv + EPS) * lncg_ref[...] + lncb_ref[...]

    tw = tokw_ref[...].astype(jnp.bfloat16)

    def proj(hp):
        t = jnp.dot(hp.astype(jnp.bfloat16), tw,
                    preferred_element_type=jnp.float32) + tokb_ref[...]
        tm = jnp.mean(t, axis=-1, keepdims=True)
        tv = jnp.mean((t - tm) ** 2, axis=-1, keepdims=True)
        t = (t - tm) * jax.lax.rsqrt(tv + EPS) * lntg_ref[...] + lntb_ref[...]
        return jnp.maximum(t, 0.0)

    r = jnp.concatenate([proj(hp) for hp in h_parts], axis=0)

    mv = m_ref[0]
    lane = jax.lax.broadcasted_iota(jnp.int32, (1, S), 1)
    L = jnp.sum(jnp.where((lane >= 1) & (lane < S - 1), mv, 0))

    q = jax.lax.broadcasted_iota(jnp.int32, (S, 1), 0)
    rm = jnp.where(q < L,
                   jnp.concatenate([r[1:], jnp.zeros((1, TD), r.dtype)], axis=0),
                   0.0)

    rp = jnp.concatenate([jnp.zeros((8, TD), rm.dtype), rm], axis=0)
    t2 = rp + _shift(rp, 1)
    t4 = t2 + _shift(t2, 2)
    t8 = t4 + _shift(t4, 4)
    ws = (_shift(t8, -W) + _shift(t2, W - 2))[8:]

    n2 = jnp.sum(ws * ws, axis=-1, keepdims=True)
    scale = jnp.where((q < L) & (n2 > 0), jax.lax.rsqrt(n2), 0.0)
    reps_ref[0] = (ws * scale)[:S - 2]


def kernel(hidden, attention_mask, tok_w, tok_b, cls_w, cls_b,
           ln_tok_g, ln_tok_b, ln_cls_g, ln_cls_b):
    B, S, H = hidden.shape
    TD = tok_w.shape[1]
    CD = cls_w.shape[1]

    mask3 = attention_mask.reshape(B, 1, S)
    full = lambda shape: pl.BlockSpec(shape, lambda b: (0,) * len(shape))

    cls3, reps = pl.pallas_call(
        _body,
        grid=(B,),
        in_specs=[
            pl.BlockSpec((1, S // 8, H), lambda b: (b, 0, 0)),
            pl.BlockSpec((1, S // 8, H), lambda b: (b, 1, 0)),
            pl.BlockSpec((1, S // 8, H), lambda b: (b, 2, 0)),
            pl.BlockSpec((1, S // 8, H), lambda b: (b, 3, 0)),
            pl.BlockSpec((1, S // 8, H), lambda b: (b, 4, 0)),
            pl.BlockSpec((1, S // 8, H), lambda b: (b, 5, 0)),
            pl.BlockSpec((1, S // 8, H), lambda b: (b, 6, 0)),
            pl.BlockSpec((1, S // 8, H), lambda b: (b, 7, 0)),
            pl.BlockSpec((1, 1, S), lambda b: (b, 0, 0)),
            full((H, TD)),
            full((1, TD)),
            full((H, CD)),
            full((1, CD)),
            full((1, TD)),
            full((1, TD)),
            full((1, CD)),
            full((1, CD)),
        ],
        out_specs=[
            pl.BlockSpec((1, 1, CD), lambda b: (b, 0, 0)),
            pl.BlockSpec((1, S - 2, TD), lambda b: (b, 0, 0)),
        ],
        out_shape=[
            jax.ShapeDtypeStruct((B, 1, CD), jnp.float32),
            jax.ShapeDtypeStruct((B, S - 2, TD), jnp.float32),
        ],
        compiler_params=pltpu.CompilerParams(
            dimension_semantics=("parallel",),
        ),
        name="coil_core_fused",
    )(hidden, hidden, hidden, hidden, hidden, hidden, hidden, hidden,
      mask3, tok_w, tok_b.reshape(1, TD),
      cls_w, cls_b.reshape(1, CD),
      ln_tok_g.reshape(1, TD), ln_tok_b.reshape(1, TD),
      ln_cls_g.reshape(1, CD), ln_cls_b.reshape(1, CD))

    return (cls3.reshape(B, CD), reps)
